# baseline (speedup 1.0000x reference)
import functools

import jax
import jax.numpy as jnp
from jax import lax
from jax.experimental import pallas as pl
from jax.experimental.pallas import tpu as pltpu
from jax.experimental.pallas import tpu_sc as plsc

MAX_LENGTH = 1000

NUM_CORES = 2
NUM_SUBCORES = 16
NUM_WORKERS = NUM_CORES * NUM_SUBCORES


_FLIP_BLOCK = 8


def _flip_body(w_ref, o_ref):
    for k in range(_FLIP_BLOCK):
        o_ref[k, :] = w_ref[_FLIP_BLOCK - 1 - k, :]


def _flip_rows(w):
    rows, dim = w.shape
    nblk = rows // _FLIP_BLOCK
    return pl.pallas_call(
        _flip_body,
        grid=(nblk,),
        in_specs=[pl.BlockSpec((_FLIP_BLOCK, dim), lambda b: (nblk - 1 - b, 0))],
        out_specs=pl.BlockSpec((_FLIP_BLOCK, dim), lambda b: (b, 0)),
        out_shape=jax.ShapeDtypeStruct(w.shape, w.dtype),
    )(w)


def _make_sc_kernel(seq_len: int, dim: int, dtype):
    i_per_worker = seq_len // NUM_WORKERS
    mesh = plsc.VectorSubcoreMesh(
        core_axis_name="c", subcore_axis_name="s",
        num_cores=NUM_CORES, num_subcores=NUM_SUBCORES,
    )

    @functools.partial(
        pl.kernel,
        out_type=jax.ShapeDtypeStruct((seq_len, seq_len, dim), dtype),
        mesh=mesh,
        scratch_types=[pltpu.SemaphoreType.DMA],
        compiler_params=pltpu.CompilerParams(use_tc_tiling_on_sc=False),
    )
    def sc_copy(wf_hbm, out_hbm, sem):
        wid = lax.axis_index("s") * NUM_CORES + lax.axis_index("c")
        i0 = wid * i_per_worker
        copies = []
        for t in range(i_per_worker):
            i = i0 + t
            copies.append(pltpu.async_copy(
                wf_hbm.at[pl.ds(MAX_LENGTH - 1 - i, seq_len), :],
                out_hbm.at[i],
                sem,
            ))
        for cp in copies:
            cp.wait()

    return sc_copy


def kernel(hidden_states, pe_k_weight):
    seq_len = hidden_states.shape[1]
    dim = pe_k_weight.shape[1]
    wf = _flip_rows(pe_k_weight)
    return _make_sc_kernel(seq_len, dim, pe_k_weight.dtype)(wf)


# speedup vs baseline: 16.6353x; 16.6353x over previous
import functools

import jax
import jax.numpy as jnp
from jax import lax
from jax.experimental import pallas as pl
from jax.experimental.pallas import tpu as pltpu
from jax.experimental.pallas import tpu_sc as plsc

MAX_LENGTH = 1000

NUM_CORES = 2
NUM_SUBCORES = 16
NUM_WORKERS = NUM_CORES * NUM_SUBCORES


_FLIP_BLOCK = 8


def _flip_body(w_ref, o_ref):
    for k in range(_FLIP_BLOCK):
        o_ref[k, :] = w_ref[_FLIP_BLOCK - 1 - k, :]


def _flip_rows(w):
    rows, dim = w.shape
    nblk = rows // _FLIP_BLOCK
    return pl.pallas_call(
        _flip_body,
        grid=(nblk,),
        in_specs=[pl.BlockSpec((_FLIP_BLOCK, dim), lambda b: (nblk - 1 - b, 0))],
        out_specs=pl.BlockSpec((_FLIP_BLOCK, dim), lambda b: (b, 0)),
        out_shape=jax.ShapeDtypeStruct(w.shape, w.dtype),
    )(w)


def _make_sc_kernel(seq_len: int, dim: int, dtype):
    i_per_worker = seq_len // NUM_WORKERS
    mesh = plsc.VectorSubcoreMesh(
        core_axis_name="c", subcore_axis_name="s",
        num_cores=NUM_CORES, num_subcores=NUM_SUBCORES,
    )

    chunk = seq_len // 2
    buf_rows = chunk + i_per_worker - 1

    @functools.partial(
        pl.kernel,
        out_type=jax.ShapeDtypeStruct((seq_len, seq_len, dim), dtype),
        mesh=mesh,
        scratch_types=[
            pltpu.VMEM((buf_rows, dim), dtype),
            pltpu.SemaphoreType.DMA,
        ],
        compiler_params=pltpu.CompilerParams(use_tc_tiling_on_sc=False),
    )
    def sc_copy(wf_hbm, out_hbm, buf, sem):
        wid = lax.axis_index("s") * NUM_CORES + lax.axis_index("c")
        i0 = wid * i_per_worker
        for h in range(2):
            base = MAX_LENGTH - 1 - i0 - (i_per_worker - 1) + chunk * h
            pltpu.sync_copy(wf_hbm.at[pl.ds(base, buf_rows), :], buf)
            stores = []
            for di in range(i_per_worker):
                stores.append(pltpu.async_copy(
                    buf.at[pl.ds(i_per_worker - 1 - di, chunk), :],
                    out_hbm.at[i0 + di, pl.ds(chunk * h, chunk), :],
                    sem,
                ))
            for cp in stores:
                cp.wait()

    return sc_copy


def kernel(hidden_states, pe_k_weight):
    seq_len = hidden_states.shape[1]
    dim = pe_k_weight.shape[1]
    wf = _flip_rows(pe_k_weight)
    return _make_sc_kernel(seq_len, dim, pe_k_weight.dtype)(wf)


# speedup vs baseline: 20.9970x; 1.2622x over previous
import functools

import jax
import jax.numpy as jnp
from jax import lax
from jax.experimental import pallas as pl
from jax.experimental.pallas import tpu as pltpu
from jax.experimental.pallas import tpu_sc as plsc

MAX_LENGTH = 1000
LANES = 16

NUM_CORES = 2
NUM_SUBCORES = 16
NUM_WORKERS = NUM_CORES * NUM_SUBCORES


def _make_sc_kernel(seq_len: int, dim: int, dtype):
    i_per_worker = seq_len // NUM_WORKERS
    chunk = seq_len // 2
    buf_rows = chunk + i_per_worker - 1
    vecs_per_row = dim // LANES

    mesh = plsc.VectorSubcoreMesh(
        core_axis_name="c", subcore_axis_name="s",
        num_cores=NUM_CORES, num_subcores=NUM_SUBCORES,
    )

    @functools.partial(
        pl.kernel,
        out_type=jax.ShapeDtypeStruct((seq_len, seq_len, dim), dtype),
        mesh=mesh,
        scratch_types=[
            pltpu.VMEM((buf_rows, dim), dtype),
            pltpu.SemaphoreType.DMA,
        ],
        compiler_params=pltpu.CompilerParams(use_tc_tiling_on_sc=False),
    )
    def sc_copy(w_hbm, out_hbm, buf, sem):
        wid = lax.axis_index("s") * NUM_CORES + lax.axis_index("c")
        i0 = wid * i_per_worker
        for h in range(2):
            lo = MAX_LENGTH + i0 - chunk * h - (chunk - 1)
            pltpu.sync_copy(w_hbm.at[pl.ds(lo, buf_rows), :], buf)

            def swap_row(r, _):
                rr = buf_rows - 1 - r
                for c in range(vecs_per_row):
                    sl = pl.ds(c * LANES, LANES)
                    va = buf[r, sl]
                    vb = buf[rr, sl]
                    buf[r, sl] = vb
                    buf[rr, sl] = va
                return 0

            lax.fori_loop(0, buf_rows // 2, swap_row, 0)

            stores = []
            for di in range(i_per_worker):
                stores.append(pltpu.async_copy(
                    buf.at[pl.ds(i_per_worker - 1 - di, chunk), :],
                    out_hbm.at[i0 + di, pl.ds(chunk * h, chunk), :],
                    sem,
                ))
            for cp in stores:
                cp.wait()

    return sc_copy


def kernel(hidden_states, pe_k_weight):
    seq_len = hidden_states.shape[1]
    dim = pe_k_weight.shape[1]
    return _make_sc_kernel(seq_len, dim, pe_k_weight.dtype)(pe_k_weight)


# speedup vs baseline: 41.9330x; 1.9971x over previous
import functools

import jax
import jax.numpy as jnp
from jax import lax
from jax.experimental import pallas as pl
from jax.experimental.pallas import tpu as pltpu
from jax.experimental.pallas import tpu_sc as plsc

MAX_LENGTH = 1000
LANES = 16
TILE_S = 8
TILE_L = 128

NUM_CORES = 2
NUM_SUBCORES = 16
NUM_WORKERS = NUM_CORES * NUM_SUBCORES


def _mesh():
    return plsc.VectorSubcoreMesh(
        core_axis_name="c", subcore_axis_name="s",
        num_cores=NUM_CORES, num_subcores=NUM_SUBCORES,
    )


def _num_r(seq_len: int) -> int:
    n_jt = seq_len // TILE_S
    qchunk_jt = n_jt // 4
    rw_max = (seq_len - 2 * TILE_S) // TILE_S + qchunk_jt * 3
    needed = rw_max + qchunk_jt + 1
    return -(-needed // NUM_WORKERS) * NUM_WORKERS


def _make_encode_kernel(seq_len: int, dim: int, dtype):
    n_dt = dim // TILE_L
    num_r = _num_r(seq_len)
    r_per_worker = num_r // NUM_WORKERS
    base_hi = MAX_LENGTH + seq_len - TILE_S
    win_rows = 8 * r_per_worker + 2 * (TILE_S - 1)
    n_q = TILE_S

    @functools.partial(
        pl.kernel,
        out_type=jax.ShapeDtypeStruct((n_q, num_r, n_dt, TILE_S, TILE_L), dtype),
        mesh=_mesh(),
        scratch_types=[
            pltpu.VMEM((win_rows, dim), dtype),
            pltpu.VMEM((r_per_worker, n_dt, TILE_S, TILE_L), dtype),
        ],
        compiler_params=pltpu.CompilerParams(use_tc_tiling_on_sc=False),
    )
    def encode(w_hbm, e_hbm, lbuf, ebuf):
        wid = lax.axis_index("s") * NUM_CORES + lax.axis_index("c")
        rs = wid * r_per_worker
        ws = base_hi - 8 * (rs + r_per_worker - 1) - (TILE_S - 1)
        pltpu.sync_copy(w_hbm.at[pl.ds(ws, win_rows), :], lbuf)

        def per_q(q, _):
            off = 8 * (r_per_worker - 1) + (TILE_S - 1)
            for dr in range(r_per_worker):
                for dt in range(n_dt):
                    for s in range(TILE_S):
                        idx = off + q - 8 * dr - s
                        for c in range(TILE_L // LANES):
                            ebuf[dr, dt, s, pl.ds(c * LANES, LANES)] = (
                                lbuf[idx, pl.ds(TILE_L * dt + c * LANES, LANES)]
                            )
            pltpu.sync_copy(ebuf, e_hbm.at[q, pl.ds(rs, r_per_worker)])
            return 0

        lax.fori_loop(0, n_q, per_q, 0)

    return encode


def _make_fanout_kernel(seq_len: int, dim: int, dtype):
    n_dt = dim // TILE_L
    n_jt = seq_len // TILE_S
    i_per_worker = seq_len // NUM_WORKERS
    n_quarters = 4
    qchunk_jt = n_jt // n_quarters
    win_blocks = qchunk_jt + 1
    n_phase = TILE_S

    @functools.partial(
        pl.kernel,
        out_type=jax.ShapeDtypeStruct((seq_len, n_jt, n_dt, TILE_S, TILE_L), dtype),
        mesh=_mesh(),
        scratch_types=[
            pltpu.VMEM((2, win_blocks, n_dt, TILE_S, TILE_L), dtype),
            pltpu.SemaphoreType.DMA,
            pltpu.SemaphoreType.DMA,
            pltpu.SemaphoreType.DMA,
            pltpu.SemaphoreType.DMA,
        ],
        compiler_params=pltpu.CompilerParams(use_tc_tiling_on_sc=False),
    )
    def fanout(e_hbm, b5_hbm, win, lsem0, lsem1, ssem0, ssem1):
        lsems = (lsem0, lsem1)
        ssems = (ssem0, ssem1)
        wid = lax.axis_index("s") * NUM_CORES + lax.axis_index("c")
        i0 = wid * i_per_worker
        rounds = []
        for c in range(n_quarters):
            for p in range(n_phase):
                rounds.append((c, p))

        def load(n, slot):
            c, p = rounds[n]
            rw = (seq_len - 2 * TILE_S) // TILE_S - 2 * wid + qchunk_jt * c
            return pltpu.async_copy(
                e_hbm.at[p, pl.ds(rw, win_blocks)], win.at[slot], lsems[slot])

        def stores(n, slot):
            c, p = rounds[n]
            jt0 = qchunk_jt * c
            s1 = pltpu.async_copy(
                win.at[slot, pl.ds(0, qchunk_jt)],
                b5_hbm.at[i0 + p + TILE_S, pl.ds(jt0, qchunk_jt)], ssems[slot])
            s2 = pltpu.async_copy(
                win.at[slot, pl.ds(1, qchunk_jt)],
                b5_hbm.at[i0 + p, pl.ds(jt0, qchunk_jt)], ssems[slot])
            return (s1, s2)

        n_rounds = len(rounds)
        pending_loads = [None, None]
        pending_stores = [None, None]
        pending_loads[0] = load(0, 0)
        for n in range(n_rounds):
            slot = n % 2
            pending_loads[slot].wait()
            st = stores(n, slot)
            pending_stores[slot] = st
            nxt = n + 1
            if nxt < n_rounds:
                other = nxt % 2
                if pending_stores[other] is not None:
                    pending_stores[other][0].wait()
                    pending_stores[other][1].wait()
                    pending_stores[other] = None
                pending_loads[other] = load(nxt, other)
        last = (n_rounds - 1) % 2
        pending_stores[last][0].wait()
        pending_stores[last][1].wait()

    return fanout


def kernel(hidden_states, pe_k_weight):
    seq_len = hidden_states.shape[1]
    dim = pe_k_weight.shape[1]
    dtype = pe_k_weight.dtype
    e = _make_encode_kernel(seq_len, dim, dtype)(pe_k_weight)
    b5 = _make_fanout_kernel(seq_len, dim, dtype)(e)
    out = b5.transpose(0, 1, 3, 2, 4).reshape(seq_len, seq_len, dim)
    return out


# speedup vs baseline: 59.5916x; 1.4211x over previous
import functools

import jax
import jax.numpy as jnp
from jax import lax
from jax.experimental import pallas as pl
from jax.experimental.pallas import tpu as pltpu
from jax.experimental.pallas import tpu_sc as plsc

MAX_LENGTH = 1000
LANES = 16
TILE_S = 8
TILE_L = 128

NUM_CORES = 2
NUM_SUBCORES = 16
NUM_WORKERS = NUM_CORES * NUM_SUBCORES


def _mesh():
    return plsc.VectorSubcoreMesh(
        core_axis_name="c", subcore_axis_name="s",
        num_cores=NUM_CORES, num_subcores=NUM_SUBCORES,
    )


def _num_r(seq_len: int) -> int:
    n_jt = seq_len // TILE_S
    qchunk_jt = n_jt // 4
    rw_max = (seq_len - 2 * TILE_S) // TILE_S + qchunk_jt * 3
    needed = rw_max + qchunk_jt + 1
    return -(-needed // NUM_WORKERS) * NUM_WORKERS


def _make_encode_kernel(seq_len: int, dim: int, dtype):
    n_dt = dim // TILE_L
    num_r = _num_r(seq_len)
    r_per_worker = num_r // NUM_WORKERS
    base_hi = MAX_LENGTH + seq_len - TILE_S
    win_rows = 8 * r_per_worker + 2 * (TILE_S - 1)
    n_q = TILE_S

    @functools.partial(
        pl.kernel,
        out_type=jax.ShapeDtypeStruct((n_q, num_r, n_dt, TILE_S, TILE_L), dtype),
        mesh=_mesh(),
        scratch_types=[
            pltpu.VMEM((win_rows, dim), dtype),
            pltpu.VMEM((r_per_worker, n_dt, TILE_S, TILE_L), dtype),
        ],
        compiler_params=pltpu.CompilerParams(use_tc_tiling_on_sc=False),
    )
    def encode(w_hbm, e_hbm, lbuf, ebuf):
        wid = lax.axis_index("s") * NUM_CORES + lax.axis_index("c")
        rs = wid * r_per_worker
        ws = base_hi - 8 * (rs + r_per_worker - 1) - (TILE_S - 1)
        pltpu.sync_copy(w_hbm.at[pl.ds(ws, win_rows), :], lbuf)

        def per_q(q, _):
            off = 8 * (r_per_worker - 1) + (TILE_S - 1)
            for dr in range(r_per_worker):
                for dt in range(n_dt):
                    for s in range(TILE_S):
                        idx = off + q - 8 * dr - s
                        for c in range(TILE_L // LANES):
                            ebuf[dr, dt, s, pl.ds(c * LANES, LANES)] = (
                                lbuf[idx, pl.ds(TILE_L * dt + c * LANES, LANES)]
                            )
            pltpu.sync_copy(ebuf, e_hbm.at[q, pl.ds(rs, r_per_worker)])
            return 0

        lax.fori_loop(0, n_q, per_q, 0)

    return encode


def _make_fanout_kernel(seq_len: int, dim: int, dtype):
    n_dt = dim // TILE_L
    n_jt = seq_len // TILE_S
    i_per_worker = seq_len // NUM_WORKERS
    n_rounds = 4
    ch_jt = n_jt // 8
    win_blocks = 2 * ch_jt + i_per_worker - 1
    m_groups = seq_len // TILE_S // i_per_worker

    @functools.partial(
        pl.kernel,
        out_type=jax.ShapeDtypeStruct((seq_len, n_jt, n_dt, TILE_S, TILE_L), dtype),
        mesh=_mesh(),
        scratch_types=[
            pltpu.VMEM((2, win_blocks, n_dt, TILE_S, TILE_L), dtype),
            pltpu.SemaphoreType.DMA,
            pltpu.SemaphoreType.DMA,
            pltpu.SemaphoreType.DMA,
            pltpu.SemaphoreType.DMA,
        ],
        compiler_params=pltpu.CompilerParams(use_tc_tiling_on_sc=False),
    )
    def fanout(e_hbm, b5_hbm, win, lsem0, lsem1, ssem0, ssem1):
        lsems = (lsem0, lsem1)
        ssems = (ssem0, ssem1)
        wid = lax.axis_index("s") * NUM_CORES + lax.axis_index("c")
        q = wid % TILE_S
        m0 = (wid // TILE_S) * i_per_worker

        def load(u, slot):
            rw = (n_jt - ch_jt * 2) - m0 + 2 * ch_jt * u
            return pltpu.async_copy(
                e_hbm.at[q, pl.ds(rw, win_blocks)], win.at[slot], lsems[slot])

        def stores(u, slot):
            descs = []
            for e in range(2):
                jt0 = ch_jt * (2 * u + e)
                for dm in range(i_per_worker):
                    descs.append(pltpu.async_copy(
                        win.at[slot, pl.ds(i_per_worker - 1 - dm + ch_jt * e,
                                           ch_jt)],
                        b5_hbm.at[q + TILE_S * (m0 + dm), pl.ds(jt0, ch_jt)],
                        ssems[slot]))
            return descs

        pending_loads = [None, None]
        pending_stores = [None, None]
        pending_loads[0] = load(0, 0)
        for u in range(n_rounds):
            slot = u % 2
            pending_loads[slot].wait()
            pending_stores[slot] = stores(u, slot)
            nxt = u + 1
            if nxt < n_rounds:
                other = nxt % 2
                if pending_stores[other] is not None:
                    for d in pending_stores[other]:
                        d.wait()
                    pending_stores[other] = None
                pending_loads[other] = load(nxt, other)
        for d in pending_stores[(n_rounds - 1) % 2]:
            d.wait()

    return fanout


def kernel(hidden_states, pe_k_weight):
    seq_len = hidden_states.shape[1]
    dim = pe_k_weight.shape[1]
    dtype = pe_k_weight.dtype
    e = _make_encode_kernel(seq_len, dim, dtype)(pe_k_weight)
    b5 = _make_fanout_kernel(seq_len, dim, dtype)(e)
    out = b5.transpose(0, 1, 3, 2, 4).reshape(seq_len, seq_len, dim)
    return out
